# Initial kernel scaffold; baseline (speedup 1.0000x reference)
import jax
import jax.numpy as jnp
from jax.experimental import pallas as pl


def kernel(idx, targets, table):
    raise NotImplementedError("write your pallas kernel here")


# speedup vs baseline: 1.7076x; 1.7076x over previous
import functools

import jax
import jax.numpy as jnp
from jax import lax
from jax.experimental import pallas as pl
from jax.experimental.pallas import tpu as pltpu
from jax.experimental.pallas import tpu_sc as plsc

NC = 2
NS = 16
NW = NC * NS
L = 16

VOCAB = 1000
LSE_PAD = 1024

CHUNK = 32
NBUF = 2


def _lse_body(tab_ref, lse_ref):
    x = tab_ref[...]
    m = jnp.max(x, axis=1, keepdims=True)
    s = jnp.sum(jnp.exp(x - m), axis=1, keepdims=True)
    lse_ref[...] = m + jnp.log(s)


def _row_lse(table):
    return pl.pallas_call(
        _lse_body,
        out_shape=jax.ShapeDtypeStruct((VOCAB, 1), jnp.float32),
    )(table)


def _gather_body(idx_hbm, tgt_hbm, table_hbm, lse_hbm, out_hbm, part_hbm,
                 idx_v, tgt_v, lse_v, rows_v, acc_v, gsems, ssems):
    n_per = idx_v.shape[0]
    nch = n_per // CHUNK
    rounds = nch // NBUF
    wid = lax.axis_index("s") * NC + lax.axis_index("c")
    base = wid * n_per

    pltpu.sync_copy(idx_hbm.at[pl.ds(base, n_per)], idx_v)
    pltpu.sync_copy(tgt_hbm.at[pl.ds(base, n_per)], tgt_v)
    pltpu.sync_copy(lse_hbm, lse_v)
    acc_v[...] = jnp.zeros((L,), jnp.float32)

    def start_gather(b, g):
        pltpu.make_async_copy(
            table_hbm.at[idx_v.at[pl.ds(g * CHUNK, CHUNK)]],
            rows_v.at[b], gsems[b]).start()

    def wait_gather(b, g):
        pltpu.make_async_copy(
            table_hbm.at[idx_v.at[pl.ds(g * CHUNK, CHUNK)]],
            rows_v.at[b], gsems[b]).wait()

    def start_store(b, g):
        pltpu.make_async_copy(
            rows_v.at[b], out_hbm.at[pl.ds(base + g * CHUNK, CHUNK)],
            ssems[b]).start()

    def wait_store(b, g):
        pltpu.make_async_copy(
            rows_v.at[b], out_hbm.at[pl.ds(base + g * CHUNK, CHUNK)],
            ssems[b]).wait()

    def accumulate(b, g):
        goff = g * CHUNK
        for j in range(0, CHUNK, L):
            i16 = idx_v[pl.ds(goff + j, L)]
            t16 = tgt_v[pl.ds(goff + j, L)]
            lse16 = plsc.load_gather(lse_v, [i16])
            rid = jax.lax.broadcasted_iota(jnp.int32, (L,), 0) + j
            tv16 = plsc.load_gather(rows_v.at[b], [rid, t16])
            acc_v[...] = acc_v[...] + (lse16 - tv16)

    for b in range(NBUF):
        start_gather(b, b)

    def round_body(r, carry):
        for b in range(NBUF):
            g = r * NBUF + b
            wait_gather(b, g)
            accumulate(b, g)
            start_store(b, g)
            wait_store(b, g)
            start_gather(b, g + NBUF)
        return carry

    lax.fori_loop(0, rounds - 1, round_body, 0)

    for b in range(NBUF):
        g = nch - NBUF + b
        wait_gather(b, g)
        accumulate(b, g)
        start_store(b, g)
    for b in range(NBUF):
        wait_store(b, nch - NBUF + b)

    pltpu.sync_copy(acc_v, part_hbm.at[wid])


def _sc_gather(idx_f, tgt_f, table, lse_pad):
    n = idx_f.shape[0]
    n_per = n // NW
    mesh = plsc.VectorSubcoreMesh(
        core_axis_name="c", subcore_axis_name="s",
        num_cores=NC, num_subcores=NS)
    f = pl.kernel(
        _gather_body,
        out_type=(
            jax.ShapeDtypeStruct((n, VOCAB), jnp.float32),
            jax.ShapeDtypeStruct((NW, L), jnp.float32),
        ),
        mesh=mesh,
        compiler_params=pltpu.CompilerParams(
            needs_layout_passes=False, use_tc_tiling_on_sc=False),
        scratch_types=[
            pltpu.VMEM((n_per,), jnp.int32),
            pltpu.VMEM((n_per,), jnp.int32),
            pltpu.VMEM((LSE_PAD,), jnp.float32),
            pltpu.VMEM((NBUF, CHUNK, VOCAB), jnp.float32),
            pltpu.VMEM((L,), jnp.float32),
            [pltpu.SemaphoreType.DMA] * NBUF,
            [pltpu.SemaphoreType.DMA] * NBUF,
        ],
    )
    return f(idx_f, tgt_f, table, lse_pad)


def kernel(idx, targets, table):
    idx_f = idx.reshape(-1)
    tgt_f = targets.reshape(-1)
    lse = _row_lse(table)
    lse_pad = jnp.concatenate(
        [lse[:, 0], jnp.zeros((LSE_PAD - VOCAB,), jnp.float32)])
    logits2, part = _sc_gather(idx_f, tgt_f, table, lse_pad)
    loss = jnp.sum(part) / jnp.float32(idx_f.shape[0])
    return (logits2, loss)
